# baseline (speedup 1.0000x reference)
import jax
import jax.numpy as jnp
from jax import lax
from jax.experimental import pallas as pl
from jax.experimental.pallas import tpu as pltpu
from jax.experimental.pallas import tpu_sc as plsc

_BS = 256
_NWORKERS = 32
_CHUNK = 262144 // _NWORKERS


def _tc_lse_pick_kernel(x_ref, tgt_ref, lo_ref, hi_ref, a_ref):
    x = x_ref[...]
    t2 = tgt_ref[...]
    lo = lo_ref[...].reshape(1, 64, 128)
    hi = hi_ref[...].reshape(1, 64, 128)

    xt = lax.transpose(x, (0, 2, 1))
    t3 = t2.reshape(_BS, 1, 128)

    m = jnp.max(xt, axis=1, keepdims=True)
    e = jnp.exp(xt - m)
    s = jnp.sum(e, axis=1, keepdims=True)

    onehot = (lo < t3) & ~(hi < t3)
    picked = jnp.sum(jnp.where(onehot, xt, 0.0), axis=1, keepdims=True)

    a = m + jnp.log(s) - picked
    a_ref[...] = a.reshape(_BS, 128)


def _sc_bin_kernel(t_hbm, binsp_hbm, lw_hbm, g_hbm,
                   t_v, g_v, binsp_v, lw_v):
    c = lax.axis_index("c")
    s = lax.axis_index("s")
    wid = s * 2 + c
    base = wid * _CHUNK
    pltpu.sync_copy(t_hbm.at[pl.ds(base, _CHUNK)], t_v)
    pltpu.sync_copy(binsp_hbm, binsp_v)
    pltpu.sync_copy(lw_hbm, lw_v)

    bv = [binsp_v[pl.ds(o * 16, 16)] for o in range(4)]
    lv = [lw_v[pl.ds(o * 16, 16)] for o in range(4)]
    bs = [bv[j // 16][j % 16] for j in range(1, 64)]
    lws = [lv[j // 16][j % 16] for j in range(64)]

    def body(i, carry):
        t = t_v[pl.ds(i * 16, 16)]
        g = jnp.full((16,), lws[0])
        z = jnp.zeros((16,), jnp.float32)
        for j in range(1, 64):
            bsp = jnp.full((16,), bs[j - 1])
            step = jnp.full((16,), lws[j] - lws[j - 1])
            g = g + jnp.where(bsp < t, step, z)
        g_v[pl.ds(i * 16, 16)] = g
        return carry

    lax.fori_loop(0, _CHUNK // 16, body, 0)
    pltpu.sync_copy(g_v, g_hbm.at[pl.ds(base, _CHUNK)])


def _combine_kernel(a_ref, g_ref, out_ref):
    out_ref[...] = a_ref[...] + g_ref[...]


@jax.jit
def kernel(outputs, target, bins):
    n, k = outputs.shape
    rows = n // 128
    grid = rows // _BS

    inf = jnp.inf
    lo = bins[0:64].at[0].set(-inf)
    hi = bins[1:65].at[63].set(inf)
    ones = jnp.ones((1, 128), dtype=bins.dtype)
    lo2 = lo.reshape(64, 1) * ones
    hi2 = hi.reshape(64, 1) * ones

    lw = jnp.log(bins[1:65] - bins[0:64])
    binsp = jnp.concatenate(
        [bins, jnp.full((63,), inf, dtype=bins.dtype)])
    lwp = jnp.concatenate(
        [lw, jnp.zeros((64,), jnp.float32)])

    x3 = outputs.reshape(rows, 128, k)
    t2 = target.reshape(rows, 128)

    g = pl.kernel(
        _sc_bin_kernel,
        out_type=jax.ShapeDtypeStruct((n,), jnp.float32),
        mesh=plsc.VectorSubcoreMesh(core_axis_name="c", subcore_axis_name="s"),
        scratch_types=[
            pltpu.VMEM((_CHUNK,), jnp.float32),
            pltpu.VMEM((_CHUNK,), jnp.float32),
            pltpu.VMEM((128,), jnp.float32),
            pltpu.VMEM((128,), jnp.float32),
        ],
    )(target, binsp, lwp)

    a = pl.pallas_call(
        _tc_lse_pick_kernel,
        grid=(grid,),
        in_specs=[
            pl.BlockSpec((_BS, 128, k), lambda i: (i, 0, 0)),
            pl.BlockSpec((_BS, 128), lambda i: (i, 0)),
            pl.BlockSpec((64, 128), lambda i: (0, 0)),
            pl.BlockSpec((64, 128), lambda i: (0, 0)),
        ],
        out_specs=pl.BlockSpec((_BS, 128), lambda i: (i, 0)),
        out_shape=jax.ShapeDtypeStruct((rows, 128), outputs.dtype),
    )(x3, t2, lo2, hi2)

    nll = pl.pallas_call(
        _combine_kernel,
        grid=(8,),
        in_specs=[
            pl.BlockSpec((rows // 8, 128), lambda i: (i, 0)),
            pl.BlockSpec((rows // 8, 128), lambda i: (i, 0)),
        ],
        out_specs=pl.BlockSpec((rows // 8, 128), lambda i: (i, 0)),
        out_shape=jax.ShapeDtypeStruct((rows, 128), outputs.dtype),
    )(a, g.reshape(rows, 128))
    return nll.reshape(n)


# speedup vs baseline: 1.1803x; 1.1803x over previous
import jax
import jax.numpy as jnp
from jax import lax
from jax.experimental import pallas as pl
from jax.experimental.pallas import tpu as pltpu
from jax.experimental.pallas import tpu_sc as plsc

_BS = 256
_NWORKERS = 32
_CHUNK = 262144 // _NWORKERS


def _tc_lse_pick_kernel(x_ref, tgt_ref, lo_ref, hi_ref, a_ref):
    x = x_ref[...]
    t2 = tgt_ref[...]
    lo = lo_ref[...].reshape(1, 64, 128)
    hi = hi_ref[...].reshape(1, 64, 128)

    xt = lax.transpose(x, (0, 2, 1))
    t3 = t2.reshape(_BS, 1, 128)

    m = jnp.max(xt, axis=1, keepdims=True)
    e = jnp.exp(xt - m)
    s = jnp.sum(e, axis=1, keepdims=True)

    onehot = (lo < t3) & ~(hi < t3)
    picked = jnp.sum(jnp.where(onehot, xt, 0.0), axis=1, keepdims=True)

    a = m + jnp.log(s) - picked
    a_ref[...] = a.reshape(_BS, 128)


def _sc_bin_kernel(t_hbm, bins_hbm, lw_hbm, g_hbm,
                   t_v, g_v, bins_v, lw_v):
    c = lax.axis_index("c")
    s = lax.axis_index("s")
    wid = s * 2 + c
    base = wid * _CHUNK
    pltpu.sync_copy(t_hbm.at[pl.ds(base, _CHUNK)], t_v)
    pltpu.sync_copy(bins_hbm, bins_v)
    pltpu.sync_copy(lw_hbm, lw_v)

    bv = [bins_v[pl.ds(o * 16, 16)] for o in range(4)]
    lv = [lw_v[pl.ds(o * 16, 16)] for o in range(4)]
    bs = [bv[j // 16][j % 16] for j in range(1, 64)]
    lws = [lv[j // 16][j % 16] for j in range(64)]

    def body(i, carry):
        ts = [t_v[pl.ds(i * 64 + u * 16, 16)] for u in range(4)]
        gs = [jnp.full((16,), lws[0]) for _ in range(4)]
        z = jnp.zeros((16,), jnp.float32)
        for j in range(1, 64):
            bsp = jnp.full((16,), bs[j - 1])
            step = jnp.full((16,), lws[j] - lws[j - 1])
            for u in range(4):
                gs[u] = gs[u] + jnp.where(bsp < ts[u], step, z)
        for u in range(4):
            g_v[pl.ds(i * 64 + u * 16, 16)] = gs[u]
        return carry

    lax.fori_loop(0, _CHUNK // 64, body, 0)
    pltpu.sync_copy(g_v, g_hbm.at[pl.ds(base, _CHUNK)])


def _combine_kernel(a_ref, g_ref, out_ref):
    out_ref[...] = a_ref[...] + g_ref[...]


@jax.jit
def kernel(outputs, target, bins):
    n, k = outputs.shape
    rows = n // 128
    grid = rows // _BS

    inf = jnp.inf
    lo = bins[0:64].at[0].set(-inf)
    hi = bins[1:65].at[63].set(inf)
    ones = jnp.ones((1, 128), dtype=bins.dtype)
    lo2 = lo.reshape(64, 1) * ones
    hi2 = hi.reshape(64, 1) * ones

    lw = jnp.log(bins[1:65] - bins[0:64])
    bins64 = bins[0:64]

    x3 = outputs.reshape(rows, 128, k)
    t2 = target.reshape(rows, 128)

    g = pl.kernel(
        _sc_bin_kernel,
        out_type=jax.ShapeDtypeStruct((n,), jnp.float32),
        mesh=plsc.VectorSubcoreMesh(core_axis_name="c", subcore_axis_name="s"),
        scratch_types=[
            pltpu.VMEM((_CHUNK,), jnp.float32),
            pltpu.VMEM((_CHUNK,), jnp.float32),
            pltpu.VMEM((64,), jnp.float32),
            pltpu.VMEM((64,), jnp.float32),
        ],
    )(target, bins64, lw)

    a = pl.pallas_call(
        _tc_lse_pick_kernel,
        grid=(grid,),
        in_specs=[
            pl.BlockSpec((_BS, 128, k), lambda i: (i, 0, 0)),
            pl.BlockSpec((_BS, 128), lambda i: (i, 0)),
            pl.BlockSpec((64, 128), lambda i: (0, 0)),
            pl.BlockSpec((64, 128), lambda i: (0, 0)),
        ],
        out_specs=pl.BlockSpec((_BS, 128), lambda i: (i, 0)),
        out_shape=jax.ShapeDtypeStruct((rows, 128), outputs.dtype),
    )(x3, t2, lo2, hi2)

    nll = pl.pallas_call(
        _combine_kernel,
        grid=(8,),
        in_specs=[
            pl.BlockSpec((rows // 8, 128), lambda i: (i, 0)),
            pl.BlockSpec((rows // 8, 128), lambda i: (i, 0)),
        ],
        out_specs=pl.BlockSpec((rows // 8, 128), lambda i: (i, 0)),
        out_shape=jax.ShapeDtypeStruct((rows, 128), outputs.dtype),
    )(a, g.reshape(rows, 128))
    return nll.reshape(n)


# speedup vs baseline: 1.2318x; 1.0437x over previous
import jax
import jax.numpy as jnp
from jax import lax
from jax.experimental import pallas as pl
from jax.experimental.pallas import tpu as pltpu
from jax.experimental.pallas import tpu_sc as plsc

_BS = 256
_NWORKERS = 32
_CHUNK = 262144 // _NWORKERS


def _tc_lse_pick_kernel(x_ref, tgt_ref, lo_ref, hi_ref, a_ref):
    x = x_ref[...]
    t2 = tgt_ref[...]
    lo = lo_ref[...].reshape(1, 64, 128)
    hi = hi_ref[...].reshape(1, 64, 128)

    xt = lax.transpose(x, (0, 2, 1))
    t3 = t2.reshape(_BS, 1, 128)

    m = jnp.max(xt, axis=1, keepdims=True)
    e = jnp.exp(xt - m)
    s = jnp.sum(e, axis=1, keepdims=True)

    onehot = (lo < t3) & ~(hi < t3)
    picked = jnp.sum(jnp.where(onehot, xt, 0.0), axis=1, keepdims=True)

    a = m + jnp.log(s) - picked
    a_ref[...] = a.reshape(_BS, 128)


def _sc_bin_kernel(t_hbm, bins_hbm, lw_hbm, g_hbm,
                   t_v, g_v, bins_v, lw_v):
    c = lax.axis_index("c")
    s = lax.axis_index("s")
    wid = s * 2 + c
    base = wid * _CHUNK
    pltpu.sync_copy(t_hbm.at[pl.ds(base, _CHUNK)], t_v)
    pltpu.sync_copy(bins_hbm, bins_v)
    pltpu.sync_copy(lw_hbm, lw_v)

    bv = [bins_v[pl.ds(o * 16, 16)] for o in range(4)]
    lv = [lw_v[pl.ds(o * 16, 16)] for o in range(4)]
    bs = [bv[j // 16][j % 16] for j in range(1, 64)]
    lws = [lv[j // 16][j % 16] for j in range(64)]

    def body(i, carry):
        ts = [t_v[pl.ds(i * 128 + u * 16, 16)] for u in range(8)]
        gs = [jnp.full((16,), lws[0]) for _ in range(8)]
        z = jnp.zeros((16,), jnp.float32)
        for j in range(1, 64):
            bsp = jnp.full((16,), bs[j - 1])
            step = jnp.full((16,), lws[j] - lws[j - 1])
            for u in range(8):
                gs[u] = gs[u] + jnp.where(bsp < ts[u], step, z)
        for u in range(8):
            g_v[pl.ds(i * 128 + u * 16, 16)] = gs[u]
        return carry

    lax.fori_loop(0, _CHUNK // 128, body, 0)
    pltpu.sync_copy(g_v, g_hbm.at[pl.ds(base, _CHUNK)])


def _combine_kernel(a_ref, g_ref, out_ref):
    out_ref[...] = a_ref[...] + g_ref[...]


@jax.jit
def kernel(outputs, target, bins):
    n, k = outputs.shape
    rows = n // 128
    grid = rows // _BS

    inf = jnp.inf
    lo = bins[0:64].at[0].set(-inf)
    hi = bins[1:65].at[63].set(inf)
    ones = jnp.ones((1, 128), dtype=bins.dtype)
    lo2 = lo.reshape(64, 1) * ones
    hi2 = hi.reshape(64, 1) * ones

    lw = jnp.log(bins[1:65] - bins[0:64])
    bins64 = bins[0:64]

    x3 = outputs.reshape(rows, 128, k)
    t2 = target.reshape(rows, 128)

    a = pl.pallas_call(
        _tc_lse_pick_kernel,
        grid=(grid,),
        in_specs=[
            pl.BlockSpec((_BS, 128, k), lambda i: (i, 0, 0)),
            pl.BlockSpec((_BS, 128), lambda i: (i, 0)),
            pl.BlockSpec((64, 128), lambda i: (0, 0)),
            pl.BlockSpec((64, 128), lambda i: (0, 0)),
        ],
        out_specs=pl.BlockSpec((_BS, 128), lambda i: (i, 0)),
        out_shape=jax.ShapeDtypeStruct((rows, 128), outputs.dtype),
    )(x3, t2, lo2, hi2)

    g = pl.kernel(
        _sc_bin_kernel,
        out_type=jax.ShapeDtypeStruct((n,), jnp.float32),
        mesh=plsc.VectorSubcoreMesh(core_axis_name="c", subcore_axis_name="s"),
        scratch_types=[
            pltpu.VMEM((_CHUNK,), jnp.float32),
            pltpu.VMEM((_CHUNK,), jnp.float32),
            pltpu.VMEM((64,), jnp.float32),
            pltpu.VMEM((64,), jnp.float32),
        ],
    )(target, bins64, lw)

    nll = pl.pallas_call(
        _combine_kernel,
        grid=(8,),
        in_specs=[
            pl.BlockSpec((rows // 8, 128), lambda i: (i, 0)),
            pl.BlockSpec((rows // 8, 128), lambda i: (i, 0)),
        ],
        out_specs=pl.BlockSpec((rows // 8, 128), lambda i: (i, 0)),
        out_shape=jax.ShapeDtypeStruct((rows, 128), outputs.dtype),
    )(a, g.reshape(rows, 128))
    return nll.reshape(n)


# speedup vs baseline: 1.2433x; 1.0093x over previous
import jax
import jax.numpy as jnp
from jax import lax
from jax.experimental import pallas as pl
from jax.experimental.pallas import tpu as pltpu
from jax.experimental.pallas import tpu_sc as plsc

_BS = 256
_NWORKERS = 32
_CHUNK = 262144 // _NWORKERS


def _tc_lse_pick_kernel(x_ref, tgt_ref, lo_ref, hi_ref, a_ref):
    x = x_ref[...]
    t2 = tgt_ref[...]
    lo = lo_ref[...].reshape(1, 64, 128)
    hi = hi_ref[...].reshape(1, 64, 128)

    xt = lax.transpose(x, (0, 2, 1))
    t3 = t2.reshape(_BS, 1, 128)

    m = jnp.max(xt, axis=1, keepdims=True)
    e = jnp.exp(xt - m)
    s = jnp.sum(e, axis=1, keepdims=True)

    onehot = (lo < t3) & ~(hi < t3)
    picked = jnp.sum(jnp.where(onehot, xt, 0.0), axis=1, keepdims=True)

    a = m + jnp.log(s) - picked
    a_ref[...] = a.reshape(_BS, 128)


def _sc_bin_kernel(t_hbm, bins_hbm, lw_hbm, g_hbm,
                   t_v, g_v, bins_v, lw_v):
    c = lax.axis_index("c")
    s = lax.axis_index("s")
    wid = s * 2 + c
    base = wid * _CHUNK
    pltpu.sync_copy(t_hbm.at[pl.ds(base, _CHUNK)], t_v)
    pltpu.sync_copy(bins_hbm, bins_v)
    pltpu.sync_copy(lw_hbm, lw_v)

    bv = [bins_v[pl.ds(o * 16, 16)] for o in range(4)]
    lv = [lw_v[pl.ds(o * 16, 16)] for o in range(4)]
    bs = [bv[j // 16][j % 16] for j in range(1, 64)]
    lws = [lv[j // 16][j % 16] for j in range(64)]

    def body(i, carry):
        ts = [t_v[pl.ds(i * 256 + u * 16, 16)] for u in range(16)]
        gs = [jnp.full((16,), lws[0]) for _ in range(16)]
        z = jnp.zeros((16,), jnp.float32)
        for j in range(1, 64):
            bsp = jnp.full((16,), bs[j - 1])
            step = jnp.full((16,), lws[j] - lws[j - 1])
            for u in range(16):
                gs[u] = gs[u] + jnp.where(bsp < ts[u], step, z)
        for u in range(16):
            g_v[pl.ds(i * 256 + u * 16, 16)] = gs[u]
        return carry

    lax.fori_loop(0, _CHUNK // 256, body, 0)
    pltpu.sync_copy(g_v, g_hbm.at[pl.ds(base, _CHUNK)])


def _combine_kernel(a_ref, g_ref, out_ref):
    out_ref[...] = a_ref[...] + g_ref[...]


@jax.jit
def kernel(outputs, target, bins):
    n, k = outputs.shape
    rows = n // 128
    grid = rows // _BS

    inf = jnp.inf
    lo = bins[0:64].at[0].set(-inf)
    hi = bins[1:65].at[63].set(inf)
    ones = jnp.ones((1, 128), dtype=bins.dtype)
    lo2 = lo.reshape(64, 1) * ones
    hi2 = hi.reshape(64, 1) * ones

    lw = jnp.log(bins[1:65] - bins[0:64])
    bins64 = bins[0:64]

    x3 = outputs.reshape(rows, 128, k)
    t2 = target.reshape(rows, 128)

    a = pl.pallas_call(
        _tc_lse_pick_kernel,
        grid=(grid,),
        in_specs=[
            pl.BlockSpec((_BS, 128, k), lambda i: (i, 0, 0)),
            pl.BlockSpec((_BS, 128), lambda i: (i, 0)),
            pl.BlockSpec((64, 128), lambda i: (0, 0)),
            pl.BlockSpec((64, 128), lambda i: (0, 0)),
        ],
        out_specs=pl.BlockSpec((_BS, 128), lambda i: (i, 0)),
        out_shape=jax.ShapeDtypeStruct((rows, 128), outputs.dtype),
    )(x3, t2, lo2, hi2)

    g = pl.kernel(
        _sc_bin_kernel,
        out_type=jax.ShapeDtypeStruct((n,), jnp.float32),
        mesh=plsc.VectorSubcoreMesh(core_axis_name="c", subcore_axis_name="s"),
        scratch_types=[
            pltpu.VMEM((_CHUNK,), jnp.float32),
            pltpu.VMEM((_CHUNK,), jnp.float32),
            pltpu.VMEM((64,), jnp.float32),
            pltpu.VMEM((64,), jnp.float32),
        ],
    )(target, bins64, lw)

    nll = pl.pallas_call(
        _combine_kernel,
        grid=(8,),
        in_specs=[
            pl.BlockSpec((rows // 8, 128), lambda i: (i, 0)),
            pl.BlockSpec((rows // 8, 128), lambda i: (i, 0)),
        ],
        out_specs=pl.BlockSpec((rows // 8, 128), lambda i: (i, 0)),
        out_shape=jax.ShapeDtypeStruct((rows, 128), outputs.dtype),
    )(a, g.reshape(rows, 128))
    return nll.reshape(n)
